# baseline (speedup 1.0000x reference)
import jax
import jax.numpy as jnp
from jax.experimental import pallas as pl
from jax.experimental.pallas import tpu as pltpu

_B, _T, _C, _K = 16, 2048, 256, 1024
_M = 512
_NJ = _T // _M
_S = _B * _NJ


def _fused_tc(enc_ref, dec_ref, decp_ref, wpre_ref, bpre_ref, wpost_ref,
              bpost_ref, cb_ref,
              out_ref, idx_ref, cm_ref, kl_ref,
              m0_ref, m1_ref, r20_ref, r21_ref,
              c2b_ref, cnt_ref, accm_ref, accr_ref, cw_ref):
    s = pl.program_id(0)

    @pl.when(s == 0)
    def _init_consts():
        cb = cb_ref[...]
        c2col = jnp.sum(cb * cb, axis=1, keepdims=True)
        c2b_ref[...] = jnp.broadcast_to(c2col * 0.5, (_K, _M))
        cw_ref[...] = jnp.dot(cb, wpost_ref[...],
                              preferred_element_type=jnp.float32)

    def body(mc_ref, rc_ref, mv_ref, rv_ref):
        x = enc_ref[0] - dec_ref[0]
        r = jnp.dot(x, wpre_ref[...],
                    preferred_element_type=jnp.float32) + bpre_ref[...]
        dots_t = jax.lax.dot_general(cb_ref[...], r,
                                     (((1,), (1,)), ((), ())),
                                     preferred_element_type=jnp.float32)
        mc_ref[...] = dots_t - c2b_ref[...]
        rsq = r * r
        rc_ref[...] = jnp.dot(rsq, jnp.ones((_C, 1), jnp.float32),
                              preferred_element_type=jnp.float32)

        jp = (s - 1) % _NJ

        @pl.when(jp == 0)
        def _init_batch():
            cnt_ref[...] = jnp.zeros_like(cnt_ref)
            accm_ref[...] = jnp.zeros_like(accm_ref)
            accr_ref[...] = jnp.zeros_like(accr_ref)

        m_t = mv_ref[...]
        idxr = jnp.argmax(m_t, axis=0)
        maxm = jnp.max(m_t, axis=0, keepdims=True)
        accm_ref[...] += maxm
        accr_ref[...] += rv_ref[...]
        idx_ref[0, 0, :] = idxr

        krow = jax.lax.broadcasted_iota(jnp.int32, (_K, _M), 0)
        oh_t = (krow == idxr[None, :]).astype(jnp.float32)
        cnt_ref[...] += jnp.dot(oh_t, jnp.ones((_M, 1), jnp.float32),
                                preferred_element_type=jnp.float32)

        quant = jax.lax.dot_general(oh_t, cw_ref[...],
                                    (((0,), (0,)), ((), ())),
                                    preferred_element_type=jnp.float32)
        y = jnp.dot(decp_ref[0], wpost_ref[...],
                    preferred_element_type=jnp.float32)
        out_ref[0] = y + quant + bpost_ref[...]

        @pl.when(jp == _NJ - 1)
        def _finalize_batch():
            s_mind = jnp.sum(accr_ref[...]) - 2.0 * jnp.sum(accm_ref[...])
            cm_ref[...] = (s_mind * (1.0 / (_T * _C))).reshape(1, 1, 1)
            p = cnt_ref[...] * (1.0 / _T)
            kl_ref[...] = jnp.sum(p * jnp.log(p * _K + 1e-10)).reshape(1, 1, 1)

    @pl.when(s % 2 == 0)
    def _even():
        body(m0_ref, r20_ref, m1_ref, r21_ref)

    @pl.when(s % 2 == 1)
    def _odd():
        body(m1_ref, r21_ref, m0_ref, r20_ref)


def kernel(enc, dec, W_pre, b_pre, W_post, b_post, codebook):
    bpre2 = b_pre.reshape(1, _C)
    bpost2 = b_post.reshape(1, _C)

    def _cur(s):
        sc = jnp.minimum(s, _S - 1)
        return (sc // _NJ, sc % _NJ, 0)

    def _prv(s):
        sp = jnp.maximum(s - 1, 0)
        return (sp // _NJ, sp % _NJ, 0)

    out, idx3, cm3, kl3 = pl.pallas_call(
        _fused_tc,
        grid=(_S + 1,),
        in_specs=[
            pl.BlockSpec((1, _M, _C), _cur),
            pl.BlockSpec((1, _M, _C), _cur),
            pl.BlockSpec((1, _M, _C), _prv),
            pl.BlockSpec((_C, _C), lambda s: (0, 0)),
            pl.BlockSpec((1, _C), lambda s: (0, 0)),
            pl.BlockSpec((_C, _C), lambda s: (0, 0)),
            pl.BlockSpec((1, _C), lambda s: (0, 0)),
            pl.BlockSpec((_K, _C), lambda s: (0, 0)),
        ],
        out_specs=[
            pl.BlockSpec((1, _M, _C), _prv),
            pl.BlockSpec((1, 1, _M),
                         lambda s: (jnp.maximum(s - 1, 0), 0, 0)),
            pl.BlockSpec((1, 1, 1),
                         lambda s: (jnp.maximum(s - 1, 0) // _NJ, 0, 0)),
            pl.BlockSpec((1, 1, 1),
                         lambda s: (jnp.maximum(s - 1, 0) // _NJ, 0, 0)),
        ],
        out_shape=[
            jax.ShapeDtypeStruct((_B, _T, _C), jnp.float32),
            jax.ShapeDtypeStruct((_S, 1, _M), jnp.int32),
            jax.ShapeDtypeStruct((_B, 1, 1), jnp.float32),
            jax.ShapeDtypeStruct((_B, 1, 1), jnp.float32),
        ],
        scratch_shapes=[
            pltpu.VMEM((_K, _M), jnp.float32),
            pltpu.VMEM((_K, _M), jnp.float32),
            pltpu.VMEM((_M, 1), jnp.float32),
            pltpu.VMEM((_M, 1), jnp.float32),
            pltpu.VMEM((_K, _M), jnp.float32),
            pltpu.VMEM((_K, 1), jnp.float32),
            pltpu.VMEM((1, _M), jnp.float32),
            pltpu.VMEM((_M, 1), jnp.float32),
            pltpu.VMEM((_K, _C), jnp.float32),
        ],
    )(enc, dec, dec, W_pre, bpre2, W_post, bpost2, codebook)

    indices = idx3.reshape(_B, _T)
    cm = cm3.reshape(_B)
    kl = kl3.reshape(_B)
    return out, cm, cm, kl, indices


# speedup vs baseline: 1.5597x; 1.5597x over previous
import jax
import jax.numpy as jnp
from jax.experimental import pallas as pl
from jax.experimental.pallas import tpu as pltpu

_B, _T, _C, _K = 16, 2048, 256, 1024
_M = 1024
_NJ = _T // _M


def _fused_tc(enc_ref, dec_ref, wpre_ref, bpre_ref, wpost_ref, bpost_ref,
              cb_ref,
              out_ref, idx_ref, cm_ref, kl_ref,
              c2b_ref, cnt_ref, accm_ref, accr_ref, cw_ref):
    b = pl.program_id(0)
    j = pl.program_id(1)

    @pl.when((b == 0) & (j == 0))
    def _init_consts():
        cb = cb_ref[...]
        c2col = jnp.sum(cb * cb, axis=1, keepdims=True)
        c2b_ref[...] = jnp.broadcast_to(c2col * 0.5, (_K, _M))
        cw_ref[...] = jnp.dot(cb, wpost_ref[...],
                              preferred_element_type=jnp.float32)

    @pl.when(j == 0)
    def _init_batch():
        cnt_ref[...] = jnp.zeros_like(cnt_ref)
        accm_ref[...] = jnp.zeros_like(accm_ref)
        accr_ref[...] = jnp.zeros_like(accr_ref)

    x = enc_ref[0] - dec_ref[0]
    r = jnp.dot(x, wpre_ref[...],
                preferred_element_type=jnp.float32) + bpre_ref[...]
    dots_t = jax.lax.dot_general(cb_ref[...], r, (((1,), (1,)), ((), ())),
                                 preferred_element_type=jnp.float32)
    m_t = dots_t - c2b_ref[...]

    idxr = jnp.argmax(m_t, axis=0)
    maxm = jnp.max(m_t, axis=0, keepdims=True)
    rsq = r * r
    r2 = jnp.dot(rsq, jnp.ones((_C, 1), jnp.float32),
                 preferred_element_type=jnp.float32)
    accm_ref[...] += maxm
    accr_ref[...] += r2
    idx_ref[0, 0, :] = idxr

    krow = jax.lax.broadcasted_iota(jnp.int32, (_K, _M), 0)
    oh_t = (krow == idxr[None, :]).astype(jnp.float32)
    cnt_ref[...] += jnp.dot(oh_t, jnp.ones((_M, 1), jnp.float32),
                            preferred_element_type=jnp.float32)

    quant = jax.lax.dot_general(oh_t, cw_ref[...], (((0,), (0,)), ((), ())),
                                preferred_element_type=jnp.float32)
    y = jnp.dot(dec_ref[0], wpost_ref[...],
                preferred_element_type=jnp.float32)
    out_ref[0] = y + quant + bpost_ref[...]

    @pl.when(j == _NJ - 1)
    def _finalize_batch():
        s_mind = jnp.sum(accr_ref[...]) - 2.0 * jnp.sum(accm_ref[...])
        cm_ref[...] = (s_mind * (1.0 / (_T * _C))).reshape(1, 1, 1)
        p = cnt_ref[...] * (1.0 / _T)
        kl_ref[...] = jnp.sum(p * jnp.log(p * _K + 1e-10)).reshape(1, 1, 1)


def kernel(enc, dec, W_pre, b_pre, W_post, b_post, codebook):
    bpre2 = b_pre.reshape(1, _C)
    bpost2 = b_post.reshape(1, _C)

    out, idx3, cm3, kl3 = pl.pallas_call(
        _fused_tc,
        grid=(_B, _NJ),
        in_specs=[
            pl.BlockSpec((1, _M, _C), lambda b, j: (b, j, 0)),
            pl.BlockSpec((1, _M, _C), lambda b, j: (b, j, 0)),
            pl.BlockSpec((_C, _C), lambda b, j: (0, 0)),
            pl.BlockSpec((1, _C), lambda b, j: (0, 0)),
            pl.BlockSpec((_C, _C), lambda b, j: (0, 0)),
            pl.BlockSpec((1, _C), lambda b, j: (0, 0)),
            pl.BlockSpec((_K, _C), lambda b, j: (0, 0)),
        ],
        out_specs=[
            pl.BlockSpec((1, _M, _C), lambda b, j: (b, j, 0)),
            pl.BlockSpec((1, 1, _M), lambda b, j: (b * _NJ + j, 0, 0)),
            pl.BlockSpec((1, 1, 1), lambda b, j: (b, 0, 0)),
            pl.BlockSpec((1, 1, 1), lambda b, j: (b, 0, 0)),
        ],
        out_shape=[
            jax.ShapeDtypeStruct((_B, _T, _C), jnp.float32),
            jax.ShapeDtypeStruct((_B * _NJ, 1, _M), jnp.int32),
            jax.ShapeDtypeStruct((_B, 1, 1), jnp.float32),
            jax.ShapeDtypeStruct((_B, 1, 1), jnp.float32),
        ],
        scratch_shapes=[
            pltpu.VMEM((_K, _M), jnp.float32),
            pltpu.VMEM((_K, 1), jnp.float32),
            pltpu.VMEM((1, _M), jnp.float32),
            pltpu.VMEM((_M, 1), jnp.float32),
            pltpu.VMEM((_K, _C), jnp.float32),
        ],
    )(enc, dec, W_pre, bpre2, W_post, bpost2, codebook)

    indices = idx3.reshape(_B, _T)
    cm = cm3.reshape(_B)
    kl = kl3.reshape(_B)
    return out, cm, cm, kl, indices


# speedup vs baseline: 1.9609x; 1.2572x over previous
import jax
import jax.numpy as jnp
from jax.experimental import pallas as pl
from jax.experimental.pallas import tpu as pltpu

_B, _T, _C, _K = 16, 2048, 256, 1024
_M = 2048
_NJ = _T // _M


def _fused_tc(enc_ref, dec_ref, wpre_ref, bpre_ref, wpost_ref, bpost_ref,
              cb_ref,
              out_ref, idx_ref, cm_ref, kl_ref,
              c2b_ref, cnt_ref, accm_ref, accr_ref, cw_ref):
    b = pl.program_id(0)
    j = pl.program_id(1)

    @pl.when((b == 0) & (j == 0))
    def _init_consts():
        cb = cb_ref[...]
        c2col = jnp.sum(cb * cb, axis=1, keepdims=True)
        c2b_ref[...] = jnp.broadcast_to(c2col * 0.5, (_K, _M))
        cw_ref[...] = jnp.dot(cb, wpost_ref[...],
                              preferred_element_type=jnp.float32)

    @pl.when(j == 0)
    def _init_batch():
        cnt_ref[...] = jnp.zeros_like(cnt_ref)
        accm_ref[...] = jnp.zeros_like(accm_ref)
        accr_ref[...] = jnp.zeros_like(accr_ref)

    x = enc_ref[0] - dec_ref[0]
    r = jnp.dot(x, wpre_ref[...],
                preferred_element_type=jnp.float32) + bpre_ref[...]
    dots_t = jax.lax.dot_general(cb_ref[...], r, (((1,), (1,)), ((), ())),
                                 preferred_element_type=jnp.float32)
    m_t = dots_t - c2b_ref[...]

    idxr = jnp.argmax(m_t, axis=0)
    maxm = jnp.max(m_t, axis=0, keepdims=True)
    rsq = r * r
    r2 = jnp.dot(rsq, jnp.ones((_C, 1), jnp.float32),
                 preferred_element_type=jnp.float32)
    accm_ref[...] += maxm
    accr_ref[...] += r2
    idx_ref[0, 0, :] = idxr

    krow = jax.lax.broadcasted_iota(jnp.int32, (_K, _M), 0)
    oh_t = (krow == idxr[None, :]).astype(jnp.float32)
    cnt_ref[...] += jnp.dot(oh_t, jnp.ones((_M, 1), jnp.float32),
                            preferred_element_type=jnp.float32)

    quant = jax.lax.dot_general(oh_t, cw_ref[...], (((0,), (0,)), ((), ())),
                                preferred_element_type=jnp.float32)
    y = jnp.dot(dec_ref[0], wpost_ref[...],
                preferred_element_type=jnp.float32)
    out_ref[0] = y + quant + bpost_ref[...]

    @pl.when(j == _NJ - 1)
    def _finalize_batch():
        s_mind = jnp.sum(accr_ref[...]) - 2.0 * jnp.sum(accm_ref[...])
        cm_ref[...] = (s_mind * (1.0 / (_T * _C))).reshape(1, 1, 1)
        p = cnt_ref[...] * (1.0 / _T)
        kl_ref[...] = jnp.sum(p * jnp.log(p * _K + 1e-10)).reshape(1, 1, 1)


def kernel(enc, dec, W_pre, b_pre, W_post, b_post, codebook):
    bpre2 = b_pre.reshape(1, _C)
    bpost2 = b_post.reshape(1, _C)

    out, idx3, cm3, kl3 = pl.pallas_call(
        _fused_tc,
        grid=(_B, _NJ),
        in_specs=[
            pl.BlockSpec((1, _M, _C), lambda b, j: (b, j, 0)),
            pl.BlockSpec((1, _M, _C), lambda b, j: (b, j, 0)),
            pl.BlockSpec((_C, _C), lambda b, j: (0, 0)),
            pl.BlockSpec((1, _C), lambda b, j: (0, 0)),
            pl.BlockSpec((_C, _C), lambda b, j: (0, 0)),
            pl.BlockSpec((1, _C), lambda b, j: (0, 0)),
            pl.BlockSpec((_K, _C), lambda b, j: (0, 0)),
        ],
        out_specs=[
            pl.BlockSpec((1, _M, _C), lambda b, j: (b, j, 0)),
            pl.BlockSpec((1, 1, _M), lambda b, j: (b * _NJ + j, 0, 0)),
            pl.BlockSpec((1, 1, 1), lambda b, j: (b, 0, 0)),
            pl.BlockSpec((1, 1, 1), lambda b, j: (b, 0, 0)),
        ],
        out_shape=[
            jax.ShapeDtypeStruct((_B, _T, _C), jnp.float32),
            jax.ShapeDtypeStruct((_B * _NJ, 1, _M), jnp.int32),
            jax.ShapeDtypeStruct((_B, 1, 1), jnp.float32),
            jax.ShapeDtypeStruct((_B, 1, 1), jnp.float32),
        ],
        scratch_shapes=[
            pltpu.VMEM((_K, _M), jnp.float32),
            pltpu.VMEM((_K, 1), jnp.float32),
            pltpu.VMEM((1, _M), jnp.float32),
            pltpu.VMEM((_M, 1), jnp.float32),
            pltpu.VMEM((_K, _C), jnp.float32),
        ],
    )(enc, dec, W_pre, bpre2, W_post, bpost2, codebook)

    indices = idx3.reshape(_B, _T)
    cm = cm3.reshape(_B)
    kl = kl3.reshape(_B)
    return out, cm, cm, kl, indices


# speedup vs baseline: 1.9945x; 1.0172x over previous
import jax
import jax.numpy as jnp
from jax.experimental import pallas as pl
from jax.experimental.pallas import tpu as pltpu

_B, _T, _C, _K = 16, 2048, 256, 1024


def _fused_tc(enc_ref, dec_ref, wpre_ref, bpre_ref, wpost_ref, bpost_ref,
              cb_ref,
              out_ref, idx_ref, cm_ref, kl_ref,
              c2b_ref, cw_ref):
    b = pl.program_id(0)

    @pl.when(b == 0)
    def _init_consts():
        cb = cb_ref[...]
        c2col = jnp.sum(cb * cb, axis=1, keepdims=True)
        c2b_ref[...] = jnp.broadcast_to(c2col * 0.5, (_K, _T))
        cw_ref[...] = jnp.dot(cb, wpost_ref[...],
                              preferred_element_type=jnp.float32)

    x = enc_ref[0] - dec_ref[0]
    r = jnp.dot(x, wpre_ref[...],
                preferred_element_type=jnp.float32) + bpre_ref[...]
    dots_t = jax.lax.dot_general(cb_ref[...], r, (((1,), (1,)), ((), ())),
                                 preferred_element_type=jnp.float32)
    m_t = dots_t - c2b_ref[...]

    idxr = jnp.argmax(m_t, axis=0)
    maxm = jnp.max(m_t, axis=0, keepdims=True)
    rsq = r * r
    r2 = jnp.dot(rsq, jnp.ones((_C, 1), jnp.float32),
                 preferred_element_type=jnp.float32)
    idx_ref[0, 0, :] = idxr

    krow = jax.lax.broadcasted_iota(jnp.int32, (_K, _T), 0)
    oh_t = (krow == idxr[None, :]).astype(jnp.float32)
    cnt = jnp.dot(oh_t, jnp.ones((_T, 1), jnp.float32),
                  preferred_element_type=jnp.float32)

    quant = jax.lax.dot_general(oh_t, cw_ref[...], (((0,), (0,)), ((), ())),
                                preferred_element_type=jnp.float32)
    y = jnp.dot(dec_ref[0], wpost_ref[...],
                preferred_element_type=jnp.float32)
    out_ref[0] = y + quant + bpost_ref[...]

    s_mind = jnp.sum(r2) - 2.0 * jnp.sum(maxm)
    cm_ref[...] = (s_mind * (1.0 / (_T * _C))).reshape(1, 1, 1)
    p = cnt * (1.0 / _T)
    kl_ref[...] = jnp.sum(p * jnp.log(p * _K + 1e-10)).reshape(1, 1, 1)


def kernel(enc, dec, W_pre, b_pre, W_post, b_post, codebook):
    bpre2 = b_pre.reshape(1, _C)
    bpost2 = b_post.reshape(1, _C)

    out, idx3, cm3, kl3 = pl.pallas_call(
        _fused_tc,
        grid=(_B,),
        in_specs=[
            pl.BlockSpec((1, _T, _C), lambda b: (b, 0, 0)),
            pl.BlockSpec((1, _T, _C), lambda b: (b, 0, 0)),
            pl.BlockSpec((_C, _C), lambda b: (0, 0)),
            pl.BlockSpec((1, _C), lambda b: (0, 0)),
            pl.BlockSpec((_C, _C), lambda b: (0, 0)),
            pl.BlockSpec((1, _C), lambda b: (0, 0)),
            pl.BlockSpec((_K, _C), lambda b: (0, 0)),
        ],
        out_specs=[
            pl.BlockSpec((1, _T, _C), lambda b: (b, 0, 0)),
            pl.BlockSpec((1, 1, _T), lambda b: (b, 0, 0)),
            pl.BlockSpec((1, 1, 1), lambda b: (b, 0, 0)),
            pl.BlockSpec((1, 1, 1), lambda b: (b, 0, 0)),
        ],
        out_shape=[
            jax.ShapeDtypeStruct((_B, _T, _C), jnp.float32),
            jax.ShapeDtypeStruct((_B, 1, _T), jnp.int32),
            jax.ShapeDtypeStruct((_B, 1, 1), jnp.float32),
            jax.ShapeDtypeStruct((_B, 1, 1), jnp.float32),
        ],
        scratch_shapes=[
            pltpu.VMEM((_K, _T), jnp.float32),
            pltpu.VMEM((_K, _C), jnp.float32),
        ],
    )(enc, dec, W_pre, bpre2, W_post, bpost2, codebook)

    indices = idx3.reshape(_B, _T)
    cm = cm3.reshape(_B)
    kl = kl3.reshape(_B)
    return out, cm, cm, kl, indices
